# Initial kernel scaffold; baseline (speedup 1.0000x reference)
import jax
import jax.numpy as jnp
from jax.experimental import pallas as pl


def kernel(x, neighlist, xneigh, indices, atoms_per_structure, types, device, W1, b1, W2, b2, W3, b3):
    raise NotImplementedError("write your pallas kernel here")


# speedup vs baseline: 24.0428x; 24.0428x over previous
import functools
import math

import jax
import jax.numpy as jnp
from jax import lax
from jax.experimental import pallas as pl
from jax.experimental.pallas import tpu as pltpu
from jax.experimental.pallas import tpu_sc as plsc

N_NODES = 50000
N_EDGES = 1600000
N_STRUCT = 100
HID = 16

NC = 2
NS = 16
NW = NC * NS
E_PER_W = N_EDGES // NW
CH = 2000
N_CHUNKS = E_PER_W // CH

TAB = 50048
TAB_SL = TAB // NS

LANE = 3200
N_BLK = N_EDGES // LANE


def _gather_body(src_hbm, x0_hbm, x1_hbm, x2_hbm,
                 g0_hbm, g1_hbm, g2_hbm,
                 idx_v, b0, b1, b2, sem):
    wid = lax.axis_index("s") * NC + lax.axis_index("c")

    def body(i, carry):
        base = wid * E_PER_W + i * CH
        pltpu.sync_copy(src_hbm.at[pl.ds(base, CH)], idx_v)
        pltpu.async_copy(x0_hbm.at[idx_v], b0, sem).wait()
        pltpu.async_copy(x1_hbm.at[idx_v], b1, sem).wait()
        pltpu.async_copy(x2_hbm.at[idx_v], b2, sem).wait()
        pltpu.sync_copy(b0, g0_hbm.at[pl.ds(base, CH)])
        pltpu.sync_copy(b1, g1_hbm.at[pl.ds(base, CH)])
        pltpu.sync_copy(b2, g2_hbm.at[pl.ds(base, CH)])
        return carry

    lax.fori_loop(0, N_CHUNKS, body, 0)


_sc_gather = functools.partial(
    pl.kernel,
    out_type=[jax.ShapeDtypeStruct((N_EDGES,), jnp.float32)] * 3,
    mesh=plsc.VectorSubcoreMesh(core_axis_name="c", subcore_axis_name="s"),
    scratch_types=[
        pltpu.VMEM((CH,), jnp.int32),
        pltpu.VMEM((CH,), jnp.float32),
        pltpu.VMEM((CH,), jnp.float32),
        pltpu.VMEM((CH,), jnp.float32),
        pltpu.SemaphoreType.DMA,
    ],
)(_gather_body)


def _scatter_body(src_hbm, v0_hbm, v1_hbm, v2_hbm, z_hbm,
                  out_hbm,
                  idx_v, b0, b1, b2, stage, sh0, sh1, sh2, sem):
    cid = lax.axis_index("c")
    sid = lax.axis_index("s")
    wid = sid * NC + cid

    sl = pl.ds(sid * TAB_SL, TAB_SL)
    pltpu.sync_copy(z_hbm.at[sl], stage)
    pltpu.sync_copy(stage, sh0.at[sl])
    pltpu.sync_copy(stage, sh1.at[sl])
    pltpu.sync_copy(stage, sh2.at[sl])

    plsc.subcore_barrier()

    def body(i, carry):
        base = wid * E_PER_W + i * CH
        pltpu.sync_copy(src_hbm.at[pl.ds(base, CH)], idx_v)
        pltpu.sync_copy(v0_hbm.at[pl.ds(base, CH)], b0)
        pltpu.sync_copy(v1_hbm.at[pl.ds(base, CH)], b1)
        pltpu.sync_copy(v2_hbm.at[pl.ds(base, CH)], b2)
        pltpu.sync_copy(b0, sh0.at[idx_v], add=True)
        pltpu.sync_copy(b1, sh1.at[idx_v], add=True)
        pltpu.sync_copy(b2, sh2.at[idx_v], add=True)
        return carry

    lax.fori_loop(0, N_CHUNKS, body, 0)

    plsc.subcore_barrier()

    base = cid * 3 * TAB + sid * TAB_SL
    pltpu.sync_copy(sh0.at[sl], stage)
    pltpu.sync_copy(stage, out_hbm.at[pl.ds(base, TAB_SL)])
    pltpu.sync_copy(sh1.at[sl], stage)
    pltpu.sync_copy(stage, out_hbm.at[pl.ds(base + TAB, TAB_SL)])
    pltpu.sync_copy(sh2.at[sl], stage)
    pltpu.sync_copy(stage, out_hbm.at[pl.ds(base + 2 * TAB, TAB_SL)])


_sc_scatter = functools.partial(
    pl.kernel,
    out_type=jax.ShapeDtypeStruct((NC * 3 * TAB,), jnp.float32),
    mesh=plsc.VectorSubcoreMesh(core_axis_name="c", subcore_axis_name="s"),
    scratch_types=[
        pltpu.VMEM((CH,), jnp.int32),
        pltpu.VMEM((CH,), jnp.float32),
        pltpu.VMEM((CH,), jnp.float32),
        pltpu.VMEM((CH,), jnp.float32),
        pltpu.VMEM((TAB_SL,), jnp.float32),
        pltpu.VMEM_SHARED((TAB,), jnp.float32),
        pltpu.VMEM_SHARED((TAB,), jnp.float32),
        pltpu.VMEM_SHARED((TAB,), jnp.float32),
        pltpu.SemaphoreType.DMA,
    ],
)(_scatter_body)


_C = 3.0
_A = math.sqrt(2.0 / _C)
_K1 = math.pi / _C
_K2 = 2.0 * math.pi / _C
_K3 = 3.0 * math.pi / _C


def _mlp_body(g0_ref, g1_ref, g2_ref, xn0_ref, xn1_ref, xn2_ref, seg_ref,
              w1_ref, b1_ref, w2_ref, b2_ref, w3_ref, b3_ref,
              e_ref, v0_ref, v1_ref, v2_ref):
    L = LANE
    dx = (g0_ref[...] - xn0_ref[...]).reshape(1, L)
    dy = (g1_ref[...] - xn1_ref[...]).reshape(1, L)
    dz = (g2_ref[...] - xn2_ref[...]).reshape(1, L)
    r2 = dx * dx + dy * dy + dz * dz
    r = jnp.sqrt(r2)
    rinv = 1.0 / r

    s1 = jnp.sin(_K1 * r)
    s2 = jnp.sin(_K2 * r)
    s3 = jnp.sin(_K3 * r)
    c1 = jnp.cos(_K1 * r)
    c2 = jnp.cos(_K2 * r)
    c3 = jnp.cos(_K3 * r)

    basis = jnp.concatenate(
        [_A * s1 * rinv, _A * s2 * rinv, _A * s3 * rinv], axis=0)
    dbasis = jnp.concatenate(
        [_A * (_K1 * c1 - s1 * rinv) * rinv,
         _A * (_K2 * c2 - s2 * rinv) * rinv,
         _A * (_K3 * c3 - s3 * rinv) * rinv], axis=0)

    w1 = w1_ref[...]
    w2 = w2_ref[...]
    w3 = w3_ref[...]
    b1 = b1_ref[...]
    b2 = b2_ref[...]
    b3 = b3_ref[...]

    dn = (((0,), (0,)), ((), ()))
    f32 = jnp.float32

    z1 = lax.dot_general(w1, basis, dn, preferred_element_type=f32) + b1
    t1 = lax.dot_general(w1, dbasis, dn, preferred_element_type=f32)
    sig1 = 1.0 / (1.0 + jnp.exp(-z1))
    h1 = z1 * sig1
    d1 = t1 * sig1 * (1.0 + z1 * (1.0 - sig1))

    z2 = lax.dot_general(w2, h1, dn, preferred_element_type=f32) + b2
    t2 = lax.dot_general(w2, d1, dn, preferred_element_type=f32)
    sig2 = 1.0 / (1.0 + jnp.exp(-z2))
    h2 = z2 * sig2
    d2 = t2 * sig2 * (1.0 + z2 * (1.0 - sig2))

    eij = lax.dot_general(w3, h2, dn, preferred_element_type=f32) + b3
    g = lax.dot_general(w3, d2, dn, preferred_element_type=f32)

    e_scaled = 100.0 * eij
    w = (-100.0) * g * rinv
    v0_ref[...] = (w * dx).reshape(1, 1, L)
    v1_ref[...] = (w * dy).reshape(1, 1, L)
    v2_ref[...] = (w * dz).reshape(1, 1, L)

    seg = seg_ref[...].reshape(1, L)
    iota_s = lax.broadcasted_iota(jnp.int32, (N_STRUCT, L), 0)
    onehot = (iota_s == jnp.broadcast_to(seg, (N_STRUCT, L))).astype(f32)
    part = lax.dot_general(onehot, e_scaled, (((1,), (1,)), ((), ())),
                           preferred_element_type=f32)

    @pl.when(pl.program_id(0) == 0)
    def _zero():
        e_ref[...] = jnp.zeros_like(e_ref)

    e_ref[...] += part


def _tc_mlp(g0, g1, g2, xn0, xn1, xn2, seg, W1, b1, W2, b2, W3, b3):
    edge_spec = pl.BlockSpec((1, 1, LANE), lambda i: (i, 0, 0))
    full = lambda shape: pl.BlockSpec(shape, lambda i: tuple(0 for _ in shape))
    return pl.pallas_call(
        _mlp_body,
        grid=(N_BLK,),
        in_specs=[edge_spec] * 7 + [
            full((3, HID)), full((HID, 1)), full((HID, HID)),
            full((HID, 1)), full((HID, 1)), full((1, 1)),
        ],
        out_specs=[
            pl.BlockSpec((N_STRUCT, 1), lambda i: (0, 0)),
            edge_spec, edge_spec, edge_spec,
        ],
        out_shape=[
            jax.ShapeDtypeStruct((N_STRUCT, 1), jnp.float32),
            jax.ShapeDtypeStruct((N_BLK, 1, LANE), jnp.float32),
            jax.ShapeDtypeStruct((N_BLK, 1, LANE), jnp.float32),
            jax.ShapeDtypeStruct((N_BLK, 1, LANE), jnp.float32),
        ],
    )(g0, g1, g2, xn0, xn1, xn2, seg, W1, b1, W2, b2, W3, b3)


def kernel(x, neighlist, xneigh, indices, atoms_per_structure, types, device,
           W1, b1, W2, b2, W3, b3):
    src = neighlist[:, 0]
    x0, x1, x2 = x[:, 0], x[:, 1], x[:, 2]
    xn0, xn1, xn2 = xneigh[:, 0], xneigh[:, 1], xneigh[:, 2]

    g0, g1, g2 = _sc_gather(src, x0, x1, x2)

    def ev(a):
        return a.reshape(N_BLK, 1, LANE)

    etot2d, v0, v1, v2 = _tc_mlp(
        ev(g0), ev(g1), ev(g2), ev(xn0), ev(xn1), ev(xn2), ev(indices),
        W1, b1.reshape(HID, 1), W2, b2.reshape(HID, 1), W3,
        b3.reshape(1, 1))

    zeros_tab = jnp.zeros((TAB,), jnp.float32)
    ftab = _sc_scatter(src, v0.reshape(-1), v1.reshape(-1), v2.reshape(-1),
                       zeros_tab).reshape(NC, 3, TAB)

    forces = (ftab[0] + ftab[1])[:, :N_NODES].T
    return etot2d[:, 0], forces


# speedup vs baseline: 30.2230x; 1.2570x over previous
import functools
import math

import jax
import jax.numpy as jnp
from jax import lax
from jax.experimental import pallas as pl
from jax.experimental.pallas import tpu as pltpu
from jax.experimental.pallas import tpu_sc as plsc

N_NODES = 50000
N_EDGES = 1600000
N_STRUCT = 100
HID = 16

NC = 2
NS = 16
NW = NC * NS
E_PER_W = N_EDGES // NW
CH = 2000
N_CHUNKS = E_PER_W // CH

TAB = 50048
TAB_SL = TAB // NS

SUB = 8
LANE = 1000
EB = SUB * LANE
N_BLK = N_EDGES // EB
E_ROWS = N_EDGES // LANE


def _gather_body(src_hbm, x0_hbm, x1_hbm, x2_hbm,
                 g0_hbm, g1_hbm, g2_hbm,
                 idx_v, b0, b1, b2, sem):
    wid = lax.axis_index("s") * NC + lax.axis_index("c")

    def body(i, carry):
        base = wid * E_PER_W + i * CH
        pltpu.sync_copy(src_hbm.at[pl.ds(base, CH)], idx_v)
        pltpu.async_copy(x0_hbm.at[idx_v], b0, sem).wait()
        pltpu.async_copy(x1_hbm.at[idx_v], b1, sem).wait()
        pltpu.async_copy(x2_hbm.at[idx_v], b2, sem).wait()
        pltpu.sync_copy(b0, g0_hbm.at[pl.ds(base, CH)])
        pltpu.sync_copy(b1, g1_hbm.at[pl.ds(base, CH)])
        pltpu.sync_copy(b2, g2_hbm.at[pl.ds(base, CH)])
        return carry

    lax.fori_loop(0, N_CHUNKS, body, 0)


_sc_gather = functools.partial(
    pl.kernel,
    out_type=[jax.ShapeDtypeStruct((N_EDGES,), jnp.float32)] * 3,
    mesh=plsc.VectorSubcoreMesh(core_axis_name="c", subcore_axis_name="s"),
    scratch_types=[
        pltpu.VMEM((CH,), jnp.int32),
        pltpu.VMEM((CH,), jnp.float32),
        pltpu.VMEM((CH,), jnp.float32),
        pltpu.VMEM((CH,), jnp.float32),
        pltpu.SemaphoreType.DMA,
    ],
)(_gather_body)


EBIN = 128
CORE_OUT = 3 * TAB + EBIN


def _scatter_body(src_hbm, v0_hbm, v1_hbm, v2_hbm, e_hbm, seg_hbm, z_hbm,
                  out_hbm,
                  idx_v, seg_v, b0, b1, b2, be, stage,
                  sh0, sh1, sh2, she, sem):
    cid = lax.axis_index("c")
    sid = lax.axis_index("s")
    wid = sid * NC + cid

    sl = pl.ds(sid * TAB_SL, TAB_SL)
    pltpu.sync_copy(z_hbm.at[sl], stage)
    pltpu.sync_copy(stage, sh0.at[sl])
    pltpu.sync_copy(stage, sh1.at[sl])
    pltpu.sync_copy(stage, sh2.at[sl])

    @pl.when(sid == 0)
    def _init_e():
        pltpu.sync_copy(stage.at[pl.ds(0, EBIN)], she)

    plsc.subcore_barrier()

    def body(i, carry):
        base = wid * E_PER_W + i * CH
        pltpu.sync_copy(src_hbm.at[pl.ds(base, CH)], idx_v)
        pltpu.sync_copy(seg_hbm.at[pl.ds(base, CH)], seg_v)
        pltpu.sync_copy(v0_hbm.at[pl.ds(base, CH)], b0)
        pltpu.sync_copy(v1_hbm.at[pl.ds(base, CH)], b1)
        pltpu.sync_copy(v2_hbm.at[pl.ds(base, CH)], b2)
        pltpu.sync_copy(e_hbm.at[pl.ds(base, CH)], be)
        pltpu.sync_copy(b0, sh0.at[idx_v], add=True)
        pltpu.sync_copy(b1, sh1.at[idx_v], add=True)
        pltpu.sync_copy(b2, sh2.at[idx_v], add=True)
        pltpu.sync_copy(be, she.at[seg_v], add=True)
        return carry

    lax.fori_loop(0, N_CHUNKS, body, 0)

    plsc.subcore_barrier()

    base = cid * CORE_OUT + sid * TAB_SL
    pltpu.sync_copy(sh0.at[sl], stage)
    pltpu.sync_copy(stage, out_hbm.at[pl.ds(base, TAB_SL)])
    pltpu.sync_copy(sh1.at[sl], stage)
    pltpu.sync_copy(stage, out_hbm.at[pl.ds(base + TAB, TAB_SL)])
    pltpu.sync_copy(sh2.at[sl], stage)
    pltpu.sync_copy(stage, out_hbm.at[pl.ds(base + 2 * TAB, TAB_SL)])

    @pl.when(sid == 0)
    def _out_e():
        pltpu.sync_copy(she, stage.at[pl.ds(0, EBIN)])
        pltpu.sync_copy(stage.at[pl.ds(0, EBIN)],
                        out_hbm.at[pl.ds(cid * CORE_OUT + 3 * TAB, EBIN)])


_sc_scatter = functools.partial(
    pl.kernel,
    out_type=jax.ShapeDtypeStruct((NC * CORE_OUT,), jnp.float32),
    mesh=plsc.VectorSubcoreMesh(core_axis_name="c", subcore_axis_name="s"),
    scratch_types=[
        pltpu.VMEM((CH,), jnp.int32),
        pltpu.VMEM((CH,), jnp.int32),
        pltpu.VMEM((CH,), jnp.float32),
        pltpu.VMEM((CH,), jnp.float32),
        pltpu.VMEM((CH,), jnp.float32),
        pltpu.VMEM((CH,), jnp.float32),
        pltpu.VMEM((TAB_SL,), jnp.float32),
        pltpu.VMEM_SHARED((TAB,), jnp.float32),
        pltpu.VMEM_SHARED((TAB,), jnp.float32),
        pltpu.VMEM_SHARED((TAB,), jnp.float32),
        pltpu.VMEM_SHARED((EBIN,), jnp.float32),
        pltpu.SemaphoreType.DMA,
    ],
)(_scatter_body)


_C = 3.0
_A = math.sqrt(2.0 / _C)
_K1 = math.pi / _C
_K2 = 2.0 * math.pi / _C
_K3 = 3.0 * math.pi / _C


def _mlp_body(g0_ref, g1_ref, g2_ref, xn0_ref, xn1_ref, xn2_ref,
              m1_ref, b1_ref, m2_ref, b2_ref, m3_ref, b3_ref,
              v0_ref, v1_ref, v2_ref, e_ref):
    dx = g0_ref[...] - xn0_ref[...]
    dy = g1_ref[...] - xn1_ref[...]
    dz = g2_ref[...] - xn2_ref[...]
    r2 = dx * dx + dy * dy + dz * dz
    r = jnp.sqrt(r2)
    rinv = 1.0 / r

    s1 = jnp.sin(_K1 * r)
    s2 = jnp.sin(_K2 * r)
    s3 = jnp.sin(_K3 * r)
    c1 = jnp.cos(_K1 * r)
    c2 = jnp.cos(_K2 * r)
    c3 = jnp.cos(_K3 * r)

    basis = jnp.concatenate(
        [_A * s1 * rinv, _A * s2 * rinv, _A * s3 * rinv], axis=0)
    dbasis = jnp.concatenate(
        [_A * (_K1 * c1 - s1 * rinv) * rinv,
         _A * (_K2 * c2 - s2 * rinv) * rinv,
         _A * (_K3 * c3 - s3 * rinv) * rinv], axis=0)

    m1 = m1_ref[...]
    m2 = m2_ref[...]
    m3 = m3_ref[...]
    b1 = b1_ref[...]
    b2 = b2_ref[...]
    b3 = b3_ref[...]

    dn = (((1,), (0,)), ((), ()))
    f32 = jnp.float32

    z1 = lax.dot_general(m1, basis, dn, preferred_element_type=f32) + b1
    t1 = lax.dot_general(m1, dbasis, dn, preferred_element_type=f32)
    sig1 = 1.0 / (1.0 + jnp.exp(-z1))
    h1 = z1 * sig1
    d1 = t1 * sig1 * (1.0 + z1 * (1.0 - sig1))

    z2 = lax.dot_general(m2, h1, dn, preferred_element_type=f32) + b2
    t2 = lax.dot_general(m2, d1, dn, preferred_element_type=f32)
    sig2 = 1.0 / (1.0 + jnp.exp(-z2))
    h2 = z2 * sig2
    d2 = t2 * sig2 * (1.0 + z2 * (1.0 - sig2))

    eij = lax.dot_general(m3, h2, dn, preferred_element_type=f32) + b3
    g = lax.dot_general(m3, d2, dn, preferred_element_type=f32)

    w = (-100.0) * g * rinv
    v0_ref[...] = w * dx
    v1_ref[...] = w * dy
    v2_ref[...] = w * dz
    e_ref[...] = 100.0 * eij


def _tc_mlp(g0, g1, g2, xn0, xn1, xn2, M1, B1, M2, B2, M3, B3):
    edge_spec = pl.BlockSpec((SUB, LANE), lambda i: (i, 0))
    full = lambda shape: pl.BlockSpec(shape, lambda i: tuple(0 for _ in shape))
    eout = jax.ShapeDtypeStruct((E_ROWS, LANE), jnp.float32)
    return pl.pallas_call(
        _mlp_body,
        grid=(N_BLK,),
        in_specs=[edge_spec] * 6 + [
            full((HID * SUB, 3 * SUB)), full((HID * SUB, 1)),
            full((HID * SUB, HID * SUB)), full((HID * SUB, 1)),
            full((SUB, HID * SUB)), full((1, 1)),
        ],
        out_specs=[edge_spec] * 4,
        out_shape=[eout, eout, eout, eout],
    )(g0, g1, g2, xn0, xn1, xn2, M1, B1, M2, B2, M3, B3)


def kernel(x, neighlist, xneigh, indices, atoms_per_structure, types, device,
           W1, b1, W2, b2, W3, b3):
    src = neighlist[:, 0]
    x0, x1, x2 = x[:, 0], x[:, 1], x[:, 2]
    xn0, xn1, xn2 = xneigh[:, 0], xneigh[:, 1], xneigh[:, 2]

    g0, g1, g2 = _sc_gather(src, x0, x1, x2)

    def ev(a):
        return a.reshape(E_ROWS, LANE)

    eye8 = jnp.eye(SUB, dtype=jnp.float32)
    ones8 = jnp.ones((SUB, 1), jnp.float32)
    M1 = jnp.kron(W1.T, eye8)
    M2 = jnp.kron(W2.T, eye8)
    M3 = jnp.kron(W3.T, eye8)
    B1 = jnp.kron(b1.reshape(HID, 1), ones8)
    B2 = jnp.kron(b2.reshape(HID, 1), ones8)

    v0, v1, v2, eij = _tc_mlp(
        ev(g0), ev(g1), ev(g2), ev(xn0), ev(xn1), ev(xn2),
        M1, B1, M2, B2, M3, b3.reshape(1, 1))

    zeros_tab = jnp.zeros((TAB,), jnp.float32)
    out = _sc_scatter(src, v0.reshape(-1), v1.reshape(-1), v2.reshape(-1),
                      eij.reshape(-1), indices, zeros_tab)
    out = out.reshape(NC, CORE_OUT)
    acc = out[0] + out[1]
    forces = acc[:3 * TAB].reshape(3, TAB)[:, :N_NODES].T
    etot = acc[3 * TAB:3 * TAB + N_STRUCT]
    return etot, forces
